# baseline (speedup 1.0000x reference)
import functools
import math

import jax
import jax.numpy as jnp
from jax import lax
from jax.experimental import pallas as pl
from jax.experimental.pallas import tpu as pltpu
from jax.experimental.pallas import tpu_sc as plsc

_D = 128
_U = 128
_NC, _NS = 2, 16
_NW = _NC * _NS
_CH = 128


def _gather_body(n_ch, table_hbm, idx_hbm, out_hbm, idx_v, rows_v, sem):
    wid = lax.axis_index("s") * _NC + lax.axis_index("c")
    pltpu.sync_copy(idx_hbm.at[pl.ds(wid * n_ch, n_ch)], idx_v)
    for j in range(n_ch):
        pltpu.async_copy(table_hbm.at[idx_v.at[j]], rows_v, sem).wait()
        pltpu.sync_copy(rows_v, out_hbm.at[pl.ds((wid * n_ch + j) * _CH, _CH)])


def _row_gather(table, idx2d):
    g = idx2d.shape[0]
    n_ch = g // _NW
    mesh = plsc.VectorSubcoreMesh(core_axis_name="c", subcore_axis_name="s")
    run = pl.kernel(
        functools.partial(_gather_body, n_ch),
        mesh=mesh,
        out_type=jax.ShapeDtypeStruct((g * _CH, _D), jnp.float32),
        scratch_types=[
            pltpu.VMEM((n_ch, _CH), jnp.int32),
            pltpu.VMEM((_CH, _D), jnp.float32),
            pltpu.SemaphoreType.DMA,
        ],
    )
    return run(table, idx2d)


def _m_body(inv_s, ks_ref, q_ref, m_ref):
    s = lax.dot_general(ks_ref[0].astype(jnp.bfloat16),
                        q_ref[0].astype(jnp.bfloat16),
                        (((1,), (1,)), ((), ())),
                        preferred_element_type=jnp.float32)
    m_ref[0] = (jnp.max(s, axis=0, keepdims=True)
                - jnp.sum(s, axis=0, keepdims=True) * inv_s)


def _m_stat(ksub3, q3, seq_s):
    bh, l, d = q3.shape
    return pl.pallas_call(
        functools.partial(_m_body, 1.0 / seq_s),
        grid=(bh,),
        in_specs=[pl.BlockSpec((1, _U, d), lambda i: (i, 0, 0)),
                  pl.BlockSpec((1, l, d), lambda i: (i, 0, 0))],
        out_specs=pl.BlockSpec((1, 1, l), lambda i: (i, 0, 0)),
        out_shape=jax.ShapeDtypeStruct((bh, 1, l), jnp.float32),
    )(ksub3, q3)


def _topk_body(m_ref, out_ref, scr_ref):
    bh, l = scr_ref.shape
    scr_ref[...] = m_ref[...]
    col_l = lax.broadcasted_iota(jnp.int32, (bh, l), 1)
    col_u = lax.broadcasted_iota(jnp.int32, (bh, _U), 1)

    def body(t, acc):
        m = scr_ref[...]
        mx = jnp.max(m, axis=1, keepdims=True)
        idx = jnp.min(jnp.where(m == mx, col_l, l), axis=1, keepdims=True)
        scr_ref[...] = jnp.where(col_l == idx, -jnp.inf, m)
        return acc + jnp.where(col_u == t, idx, 0)

    out_ref[...] = lax.fori_loop(0, _U, body, jnp.zeros((bh, _U), jnp.int32))


def _topk(m2d):
    bh, l = m2d.shape
    return pl.pallas_call(
        _topk_body,
        out_shape=jax.ShapeDtypeStruct((bh, _U), jnp.int32),
        scratch_shapes=[pltpu.VMEM((bh, l), jnp.float32)],
    )(m2d)


def _attn_body(scale, qr_ref, k_ref, v_ref, o_ref):
    v = v_ref[0]
    s = lax.dot_general(qr_ref[0].astype(jnp.bfloat16),
                        k_ref[0].astype(jnp.bfloat16),
                        (((1,), (1,)), ((), ())),
                        preferred_element_type=jnp.float32) * scale
    mx = jnp.max(s, axis=1, keepdims=True)
    e = jnp.exp(s - mx)
    attn = e / jnp.sum(e, axis=1, keepdims=True)
    p1 = lax.dot_general(attn.astype(jnp.bfloat16), v.astype(jnp.bfloat16),
                         (((1,), (0,)), ((), ())),
                         preferred_element_type=jnp.float32)
    vsum = jnp.sum(v, axis=0, keepdims=True)
    fill = jnp.broadcast_to(vsum, (v.shape[0] - _U, v.shape[1]))
    o_ref[0] = jnp.concatenate([p1, fill], axis=0)


def _attention(qr3, k3, v3):
    bh, s, d = k3.shape
    return pl.pallas_call(
        functools.partial(_attn_body, 1.0 / math.sqrt(d)),
        grid=(bh,),
        in_specs=[pl.BlockSpec((1, _U, d), lambda i: (i, 0, 0)),
                  pl.BlockSpec((1, s, d), lambda i: (i, 0, 0)),
                  pl.BlockSpec((1, s, d), lambda i: (i, 0, 0))],
        out_specs=pl.BlockSpec((1, s, d), lambda i: (i, 0, 0)),
        out_shape=jax.ShapeDtypeStruct((bh, s, d), jnp.float32),
    )(qr3, k3, v3)


def kernel(queries, keys, values):
    b, l, h, d = queries.shape
    s = keys.shape[1]
    bh = b * h
    q3 = jnp.reshape(queries, (bh, l, d))
    k3 = jnp.reshape(keys, (bh, s, d))
    v3 = jnp.reshape(values, (bh, s, d))

    skey = jax.random.key(42)
    _, k2 = jax.random.split(skey)
    idx_k = jax.random.randint(k2, (_U,), 0, s).astype(jnp.int32)

    offs_k = jnp.arange(bh, dtype=jnp.int32)[:, None] * s
    offs_q = jnp.arange(bh, dtype=jnp.int32)[:, None] * l
    ksub = _row_gather(jnp.reshape(k3, (bh * s, d)),
                       offs_k + idx_k[None, :])
    m3 = _m_stat(jnp.reshape(ksub, (bh, _U, d)), q3, s)
    mtop = _topk(jnp.reshape(m3, (bh, l)))
    qr = _row_gather(jnp.reshape(q3, (bh * l, d)),
                     offs_q + mtop)
    out3 = _attention(jnp.reshape(qr, (bh, _U, d)), k3, v3)
    return jnp.reshape(out3, (b, h, s, d))


# speedup vs baseline: 1.0618x; 1.0618x over previous
import functools
import math

import jax
import jax.numpy as jnp
from jax import lax
from jax.experimental import pallas as pl
from jax.experimental.pallas import tpu as pltpu
from jax.experimental.pallas import tpu_sc as plsc

_D = 128
_U = 128
_NC, _NS = 2, 16
_NW = _NC * _NS
_CH = 128


def _gather_body(n_ch, table_hbm, idx_hbm, out_hbm, idx_v, rows_v, sem):
    wid = lax.axis_index("s") * _NC + lax.axis_index("c")
    pltpu.sync_copy(idx_hbm.at[pl.ds(wid * n_ch, n_ch)], idx_v)
    for j in range(n_ch):
        pltpu.async_copy(table_hbm.at[idx_v.at[j]], rows_v, sem).wait()
        pltpu.sync_copy(rows_v, out_hbm.at[pl.ds((wid * n_ch + j) * _CH, _CH)])


def _row_gather(table, idx2d):
    g = idx2d.shape[0]
    n_ch = g // _NW
    mesh = plsc.VectorSubcoreMesh(core_axis_name="c", subcore_axis_name="s")
    run = pl.kernel(
        functools.partial(_gather_body, n_ch),
        mesh=mesh,
        out_type=jax.ShapeDtypeStruct((g * _CH, _D), jnp.float32),
        scratch_types=[
            pltpu.VMEM((n_ch, _CH), jnp.int32),
            pltpu.VMEM((_CH, _D), jnp.float32),
            pltpu.SemaphoreType.DMA,
        ],
    )
    return run(table, idx2d)


def _m_body(inv_s, ks_ref, q_ref, m_ref):
    s = lax.dot_general(ks_ref[0].astype(jnp.bfloat16),
                        q_ref[0].astype(jnp.bfloat16),
                        (((1,), (1,)), ((), ())),
                        preferred_element_type=jnp.float32)
    m_ref[0] = (jnp.max(s, axis=0, keepdims=True)
                - jnp.sum(s, axis=0, keepdims=True) * inv_s)


def _m_stat(ksub3, q3, seq_s):
    bh, l, d = q3.shape
    return pl.pallas_call(
        functools.partial(_m_body, 1.0 / seq_s),
        grid=(bh,),
        in_specs=[pl.BlockSpec((1, _U, d), lambda i: (i, 0, 0)),
                  pl.BlockSpec((1, l, d), lambda i: (i, 0, 0))],
        out_specs=pl.BlockSpec((1, 1, l), lambda i: (i, 0, 0)),
        out_shape=jax.ShapeDtypeStruct((bh, 1, l), jnp.float32),
    )(ksub3, q3)


def _topk_body(m_ref, out_ref, scr_ref):
    bh, l = scr_ref.shape
    col_l = lax.broadcasted_iota(jnp.int32, (bh, l), 1)
    col_u = lax.broadcasted_iota(jnp.int32, (bh, _U), 1)
    bits = lax.bitcast_convert_type(m_ref[...], jnp.int32)
    key = jnp.where(bits < 0, bits ^ jnp.int32(0x7FFFFFFF), bits)
    scr_ref[...] = (key & jnp.int32(~(l - 1))) | (jnp.int32(l - 1) - col_l)
    neg = jnp.int32(-(2 ** 31))

    def body(t, acc):
        k = scr_ref[...]
        mx = jnp.max(k, axis=1, keepdims=True)
        idx = jnp.int32(l - 1) - (mx & jnp.int32(l - 1))
        scr_ref[...] = jnp.where(k == mx, neg, k)
        return acc + jnp.where(col_u == t, idx, 0)

    out_ref[...] = lax.fori_loop(0, _U, body, jnp.zeros((bh, _U), jnp.int32))


def _topk(m2d):
    bh, l = m2d.shape
    return pl.pallas_call(
        _topk_body,
        out_shape=jax.ShapeDtypeStruct((bh, _U), jnp.int32),
        scratch_shapes=[pltpu.VMEM((bh, l), jnp.int32)],
    )(m2d)


def _attn_body(scale, qr_ref, k_ref, v_ref, o_ref):
    v = v_ref[0]
    s = lax.dot_general(qr_ref[0].astype(jnp.bfloat16),
                        k_ref[0].astype(jnp.bfloat16),
                        (((1,), (1,)), ((), ())),
                        preferred_element_type=jnp.float32) * scale
    mx = jnp.max(s, axis=1, keepdims=True)
    e = jnp.exp(s - mx)
    attn = e / jnp.sum(e, axis=1, keepdims=True)
    p1 = lax.dot_general(attn.astype(jnp.bfloat16), v.astype(jnp.bfloat16),
                         (((1,), (0,)), ((), ())),
                         preferred_element_type=jnp.float32)
    vsum = jnp.sum(v, axis=0, keepdims=True)
    fill = jnp.broadcast_to(vsum, (v.shape[0] - _U, v.shape[1]))
    o_ref[0] = jnp.concatenate([p1, fill], axis=0)


def _attention(qr3, k3, v3):
    bh, s, d = k3.shape
    return pl.pallas_call(
        functools.partial(_attn_body, 1.0 / math.sqrt(d)),
        grid=(bh,),
        in_specs=[pl.BlockSpec((1, _U, d), lambda i: (i, 0, 0)),
                  pl.BlockSpec((1, s, d), lambda i: (i, 0, 0)),
                  pl.BlockSpec((1, s, d), lambda i: (i, 0, 0))],
        out_specs=pl.BlockSpec((1, s, d), lambda i: (i, 0, 0)),
        out_shape=jax.ShapeDtypeStruct((bh, s, d), jnp.float32),
    )(qr3, k3, v3)


def kernel(queries, keys, values):
    b, l, h, d = queries.shape
    s = keys.shape[1]
    bh = b * h
    q3 = jnp.reshape(queries, (bh, l, d))
    k3 = jnp.reshape(keys, (bh, s, d))
    v3 = jnp.reshape(values, (bh, s, d))

    skey = jax.random.key(42)
    _, k2 = jax.random.split(skey)
    idx_k = jax.random.randint(k2, (_U,), 0, s).astype(jnp.int32)

    offs_k = jnp.arange(bh, dtype=jnp.int32)[:, None] * s
    offs_q = jnp.arange(bh, dtype=jnp.int32)[:, None] * l
    ksub = _row_gather(jnp.reshape(k3, (bh * s, d)),
                       offs_k + idx_k[None, :])
    m3 = _m_stat(jnp.reshape(ksub, (bh, _U, d)), q3, s)
    mtop = _topk(jnp.reshape(m3, (bh, l)))
    qr = _row_gather(jnp.reshape(q3, (bh * l, d)),
                     offs_q + mtop)
    out3 = _attention(jnp.reshape(qr, (bh, _U, d)), k3, v3)
    return jnp.reshape(out3, (b, h, s, d))
